# Initial kernel scaffold; baseline (speedup 1.0000x reference)
import jax
import jax.numpy as jnp
from jax.experimental import pallas as pl


def kernel(src, emb_table, pos_table):
    raise NotImplementedError("write your pallas kernel here")


# speedup vs baseline: 1.2708x; 1.2708x over previous
import functools

import jax
import jax.numpy as jnp
from jax import lax
from jax.experimental import pallas as pl
from jax.experimental.pallas import tpu as pltpu
from jax.experimental.pallas import tpu_sc as plsc

_VOCAB = 100000
_CTX = 2048
_D = 768
_BATCH = 4
_SEQ = 2048

_NC = 2
_NS = 16
_NW = _NC * _NS
_N = _BATCH * _SEQ
_PER_W = _N // _NW
_C = 32
_NCHUNK = _PER_W // _C
_LANES = 16


def _body(src_hbm, pos_hbm, emb_hbm, out_hbm,
          idx_v, rows0, rows1, pos0, pos1,
          gsem0, gsem1, psem0, psem1):
    wid = lax.axis_index("s") * _NC + lax.axis_index("c")
    base = wid * _PER_W
    pos_base = lax.rem(base, _SEQ)

    rows_bufs = [rows0, rows1]
    pos_bufs = [pos0, pos1]
    gsems = [gsem0, gsem1]
    psems = [psem0, psem1]

    pltpu.sync_copy(src_hbm.at[wid], idx_v)

    def issue(c):
        nb = c % 2
        pltpu.async_copy(emb_hbm.at[idx_v.at[c]], rows_bufs[nb], gsems[nb])
        pltpu.async_copy(pos_hbm.at[pl.ds(pos_base + c * _C, _C)],
                         pos_bufs[nb], psems[nb])

    issue(0)
    for c in range(_NCHUNK):
        nb = c % 2
        pltpu.make_async_copy(emb_hbm.at[idx_v.at[c]], rows_bufs[nb],
                              gsems[nb]).wait()
        pltpu.make_async_copy(pos_hbm.at[pl.ds(pos_base + c * _C, _C)],
                              pos_bufs[nb], psems[nb]).wait()
        if c + 1 < _NCHUNK:
            issue(c + 1)

        rows = rows_bufs[nb]
        pos = pos_bufs[nb]

        def row_body(r, carry):
            for j in range(_D // _LANES):
                s = pl.ds(j * _LANES, _LANES)
                rows[r, s] = rows[r, s] + pos[r, s]
            return carry

        lax.fori_loop(0, _C, row_body, 0)

        pltpu.sync_copy(rows, out_hbm.at[pl.ds(base + c * _C, _C)])


@jax.jit
def _embed(src_flat, emb_table, pos_table):
    kfn = pl.kernel(
        _body,
        out_type=jax.ShapeDtypeStruct((_N, _D), jnp.float32),
        mesh=plsc.VectorSubcoreMesh(core_axis_name="c", subcore_axis_name="s",
                                    num_cores=_NC, num_subcores=_NS),
        scratch_types=[
            pltpu.VMEM((_NCHUNK, _C), jnp.int32),
            pltpu.VMEM((_C, _D), jnp.float32),
            pltpu.VMEM((_C, _D), jnp.float32),
            pltpu.VMEM((_C, _D), jnp.float32),
            pltpu.VMEM((_C, _D), jnp.float32),
            pltpu.SemaphoreType.DMA,
            pltpu.SemaphoreType.DMA,
            pltpu.SemaphoreType.DMA,
            pltpu.SemaphoreType.DMA,
        ],
    )
    return kfn(src_flat, pos_table, emb_table)


def kernel(src, emb_table, pos_table):
    batch, seq = src.shape
    src_flat = src.reshape(_NW, _NCHUNK, _C).astype(jnp.int32)
    out = _embed(src_flat, emb_table, pos_table)
    return out.reshape(batch, seq, _D)
